# Initial kernel scaffold; baseline (speedup 1.0000x reference)
import jax
import jax.numpy as jnp
from jax.experimental import pallas as pl


def kernel(x, table):
    raise NotImplementedError("write your pallas kernel here")


# speedup vs baseline: 2.5721x; 2.5721x over previous
import functools

import jax
import jax.numpy as jnp
from jax import lax
from jax.experimental import pallas as pl
from jax.experimental.pallas import tpu as pltpu
from jax.experimental.pallas import tpu_sc as plsc

VOCAB_SIZE = 64
HIDDEN_DIM = 64

_LANE = 128
_K = 8
_TOKENS = 16384 * 200
_ROWS = _TOKENS // _LANE
_NW = 32
_ROWS_PER_W = _ROWS // _NW
_OUTER = _ROWS_PER_W // _K


def _emb_body(x_hbm, table_hbm, out_hbm, idx_v, rows_v, sem):
    wid = lax.axis_index("s") * 2 + lax.axis_index("c")
    w_base = wid * _ROWS_PER_W

    def body(it, carry):
        base = w_base + it * _K
        pltpu.sync_copy(x_hbm.at[pl.ds(base, _K)], idx_v)
        copies = []
        for j in range(_K):
            copies.append(
                pltpu.async_copy(table_hbm.at[idx_v.at[j]], rows_v.at[j], sem)
            )
        for c in copies:
            c.wait()
        pltpu.sync_copy(rows_v, out_hbm.at[pl.ds(base, _K)])
        return carry

    lax.fori_loop(0, _OUTER, body, 0)


def kernel(x, table):
    x2 = x.reshape(_ROWS, _LANE).astype(jnp.int32)
    mesh = plsc.VectorSubcoreMesh(core_axis_name="c", subcore_axis_name="s")
    run = functools.partial(
        pl.kernel,
        mesh=mesh,
        out_type=jax.ShapeDtypeStruct((_ROWS, _LANE, HIDDEN_DIM), jnp.float32),
        scratch_types=[
            pltpu.VMEM((_K, _LANE), jnp.int32),
            pltpu.VMEM((_K, _LANE, HIDDEN_DIM), jnp.float32),
            pltpu.SemaphoreType.DMA,
        ],
        compiler_params=pltpu.CompilerParams(use_tc_tiling_on_sc=False),
    )(_emb_body)
    out = run(x2, table)
    return out.reshape(16384, 200, HIDDEN_DIM)
